# baseline (speedup 1.0000x reference)
import functools
import math

import jax
import jax.numpy as jnp
from jax import lax
from jax.experimental import pallas as pl
from jax.experimental.pallas import tpu as pltpu
from jax.experimental.pallas import tpu_sc as plsc

VOCAB = 1_000_000
D_MODEL = 64
SCALE = math.sqrt(D_MODEL)

NUM_CORES = 2
NUM_SUBCORES = 16
LANES = 16
NUM_WORKERS = NUM_CORES * NUM_SUBCORES

NBLK = 7813
FULL_BLK = 7812
BLK_PER_W = 245
PAIR_ROWS = VOCAB // 2

_MESH = dict(core_axis_name="c", subcore_axis_name="s",
             num_cores=NUM_CORES, num_subcores=NUM_SUBCORES)


def _worker_id():
  return lax.axis_index("s") * NUM_CORES + lax.axis_index("c")


@functools.partial(
    pl.kernel,
    out_type=jax.ShapeDtypeStruct((PAIR_ROWS, 128), jnp.float32),
    mesh=plsc.VectorSubcoreMesh(**_MESH),
    scratch_types=[
        pltpu.VMEM((2, 64, 128), jnp.float32),
        pltpu.VMEM((2, 64, 128), jnp.float32),
        pltpu.SemaphoreType.DMA((2,)),
        pltpu.SemaphoreType.DMA((2,)),
    ],
    compiler_params=pltpu.CompilerParams(use_tc_tiling_on_sc=True, needs_layout_passes=False),
)
def _convert(tt_hbm, tailp_hbm, tp_hbm, src, dst, isem, osem):
  w = _worker_id()
  riota = lax.iota(jnp.int32, LANES)
  rows_m = [riota + 16 * (m % 4) for m in range(8)]

  @pl.when(w == 0)
  def _tail():
    pltpu.sync_copy(tailp_hbm, src.at[0].at[pl.ds(0, 32)])
    pltpu.sync_copy(src.at[0].at[pl.ds(0, 32)],
                    tp_hbm.at[pl.ds(FULL_BLK * 64, 32)])

  def transpose_block(b):
    @pl.loop(0, 64)
    def _p(p):
      for m in range(8):
        e = m // 4
        col = jnp.full((LANES,), 2 * p + e, jnp.int32)
        val = plsc.load_gather(src.at[b], [rows_m[m], col])
        dst[b, p, pl.ds(16 * m, LANES)] = val

  @pl.loop(0, BLK_PER_W + 1, step=2)
  def _group(t0):
    for b in range(2):
      c = w + NUM_WORKERS * (t0 + b)

      @pl.when(c < FULL_BLK)
      def _fire(b=b, c=c):
        pltpu.async_copy(
            tt_hbm.at[:, pl.ds(c * 128, 128)], src.at[b], isem.at[b])

    for b in range(2):
      c = w + NUM_WORKERS * (t0 + b)

      @pl.when(c < FULL_BLK)
      def _do(b=b, c=c):
        pltpu.make_async_copy(
            tt_hbm.at[:, pl.ds(c * 128, 128)], src.at[b], isem.at[b]).wait()
        transpose_block(b)
        pltpu.async_copy(
            dst.at[b], tp_hbm.at[pl.ds(c * 64, 64)], osem.at[b])

    for b in range(2):
      c = w + NUM_WORKERS * (t0 + b)

      @pl.when(c < FULL_BLK)
      def _drain(b=b, c=c):
        pltpu.make_async_copy(
            dst.at[b], tp_hbm.at[pl.ds(c * 64, 64)], osem.at[b]).wait()


COLS_PER_W = 6400 // NUM_WORKERS


@functools.partial(
    pl.kernel,
    out_type=jax.ShapeDtypeStruct((200, 8, 32, 8, 128), jnp.float32),
    mesh=plsc.VectorSubcoreMesh(**_MESH),
    scratch_types=[
        pltpu.VMEM((2, 128), jnp.int32),
        pltpu.VMEM((2, 128), jnp.int32),
        pltpu.VMEM((2, 128), jnp.int32),
        pltpu.VMEM((2, 128, 128), jnp.float32),
        pltpu.VMEM((2, 64, 128), jnp.float32),
        pltpu.SemaphoreType.DMA((2,)),
        pltpu.SemaphoreType.DMA((2,)),
        pltpu.SemaphoreType.DMA((2,)),
    ],
    compiler_params=pltpu.CompilerParams(use_tc_tiling_on_sc=True, needs_layout_passes=False),
)
def _lookup(tp_hbm, xt_hbm, o5_hbm, idx, idxp, colb, rows, outb,
            isem, gsem, osem):
  w = _worker_id()
  t_base = w * COLS_PER_W
  riota = lax.iota(jnp.int32, LANES)
  rows_m = [riota + 16 * m for m in range(8)]

  @pl.loop(0, COLS_PER_W, step=2)
  def _group(g0):
    hs, cs = [], []
    idx_cps = []
    for b in range(2):
      t = t_base + g0 + b
      h = t >> 5
      c = t & 31
      hs.append(h)
      cs.append(c)
      idx_cps.append(pltpu.async_copy(
          xt_hbm.at[h, pl.ds(c * 128, 128)], idx.at[b], isem.at[b]))
    row_cps = []
    for b in range(2):
      idx_cps[b].wait()
      for i in range(8):
        a = idx[b, pl.ds(16 * i, LANES)]
        idxp[b, pl.ds(16 * i, LANES)] = lax.shift_right_logical(a, 1)
        colb[b, pl.ds(16 * i, LANES)] = (a & 1) * 64
      row_cps.append(pltpu.async_copy(
          tp_hbm.at[idxp.at[b]], rows.at[b], gsem.at[b]))
    out_cps = []
    for b in range(2):
      row_cps[b].wait()
      cb = [colb[b, pl.ds(16 * m, LANES)] for m in range(8)]

      @pl.loop(0, D_MODEL, unroll=2)
      def _d(d):
        for m in range(8):
          val = plsc.load_gather(rows.at[b], [rows_m[m], cb[m] + d])
          outb[b, d, pl.ds(16 * m, LANES)] = val * SCALE

      for k in range(8):
        out_cps.append(pltpu.async_copy(
            outb.at[b].at[pl.ds(8 * k, 8)],
            o5_hbm.at[hs[b], k, cs[b]], osem.at[b]))
    for cp in out_cps:
      cp.wait()


def kernel(x, table):
  xt = x.astype(jnp.int32).T
  tt = table.T
  tailp = table[FULL_BLK * 128:].reshape(32, 128)
  tp = _convert(tt, tailp)
  o5 = _lookup(tp, xt)
  return o5.transpose(2, 4, 0, 1, 3).reshape(4096, 200, 64)
